# Initial kernel scaffold; baseline (speedup 1.0000x reference)
import jax
import jax.numpy as jnp
from jax.experimental import pallas as pl


def kernel(z, codebooks):
    raise NotImplementedError("write your pallas kernel here")


# speedup vs baseline: 2.7180x; 2.7180x over previous
import functools

import jax
import jax.numpy as jnp
from jax.experimental import pallas as pl


def _vq_body(z_ref, cbT_ref, cbR_ref, out_ref, *, L, G, K, D):
    z = z_ref[...]
    res = z
    quant = jnp.zeros_like(z)
    for l in range(L):
        parts = []
        for g in range(G):
            lg = l * G + g
            ct = cbT_ref[lg]
            r = res[:, g * D:(g + 1) * D]
            dot = jax.lax.dot_general(
                r, ct, (((1,), (0,)), ((), ())),
                preferred_element_type=jnp.float32)
            cb2 = jnp.sum(ct * ct, axis=0, keepdims=True)
            r2 = jnp.sum(r * r, axis=1, keepdims=True)
            dist = r2 - 2.0 * dot + cb2
            mins = jnp.min(dist, axis=1, keepdims=True)
            iota = jax.lax.broadcasted_iota(jnp.int32, dist.shape, 1)
            idx = jnp.min(jnp.where(dist == mins, iota, K), axis=1,
                          keepdims=True)
            oh = (iota == idx).astype(jnp.float32)
            q = jax.lax.dot_general(
                oh, cbR_ref[lg], (((1,), (0,)), ((), ())),
                preferred_element_type=jnp.float32,
                precision=jax.lax.Precision.HIGHEST)
            parts.append(q)
        qfull = jnp.concatenate(parts, axis=1)
        quant = quant + qfull
        res = res - qfull
    out_ref[...] = quant


def kernel(z, codebooks):
    B, N, C = z.shape
    L, G, K, D = codebooks.shape
    T = B * N
    TT = 512
    z2 = z.reshape(T, C)
    cbT = codebooks.transpose(0, 1, 3, 2).reshape(L * G, D, K)
    cbR = codebooks.reshape(L * G, K, D)
    quant = pl.pallas_call(
        functools.partial(_vq_body, L=L, G=G, K=K, D=D),
        grid=(T // TT,),
        in_specs=[
            pl.BlockSpec((TT, C), lambda i: (i, 0)),
            pl.BlockSpec((L * G, D, K), lambda i: (0, 0, 0)),
            pl.BlockSpec((L * G, K, D), lambda i: (0, 0, 0)),
        ],
        out_specs=pl.BlockSpec((TT, C), lambda i: (i, 0)),
        out_shape=jax.ShapeDtypeStruct((T, C), z.dtype),
    )(z2, cbT, cbR)
    q = quant.reshape(B, N, C)
    return z + jax.lax.stop_gradient(q - z)


# speedup vs baseline: 3.3061x; 1.2164x over previous
import functools

import jax
import jax.numpy as jnp
from jax.experimental import pallas as pl


def _vq_body(z_ref, cbT_ref, cbR_ref, out_ref, *, L, G, K, D):
    z = z_ref[...]
    res = z
    quant = jnp.zeros_like(z)
    for l in range(L):
        parts = []
        for g in range(G):
            lg = l * G + g
            ct = cbT_ref[lg]
            r = res[:, g * D:(g + 1) * D]
            dot = jax.lax.dot_general(
                r, ct, (((1,), (0,)), ((), ())),
                preferred_element_type=jnp.float32)
            cb2 = jnp.sum(ct * ct, axis=0, keepdims=True)
            r2 = jnp.sum(r * r, axis=1, keepdims=True)
            dist = r2 - 2.0 * dot + cb2
            mins = jnp.min(dist, axis=1, keepdims=True)
            iota = jax.lax.broadcasted_iota(jnp.int32, dist.shape, 1)
            idx = jnp.min(jnp.where(dist == mins, iota, K), axis=1,
                          keepdims=True)
            oh = (iota == idx).astype(jnp.float32)
            q = jax.lax.dot_general(
                oh, cbR_ref[lg], (((1,), (0,)), ((), ())),
                preferred_element_type=jnp.float32,
                precision=jax.lax.Precision.HIGHEST)
            parts.append(q)
        qfull = jnp.concatenate(parts, axis=1)
        quant = quant + qfull
        res = res - qfull
    out_ref[...] = quant


def kernel(z, codebooks):
    B, N, C = z.shape
    L, G, K, D = codebooks.shape
    T = B * N
    TT = 1024
    z2 = z.reshape(T, C)
    cbT = codebooks.transpose(0, 1, 3, 2).reshape(L * G, D, K)
    cbR = codebooks.reshape(L * G, K, D)
    quant = pl.pallas_call(
        functools.partial(_vq_body, L=L, G=G, K=K, D=D),
        grid=(T // TT,),
        in_specs=[
            pl.BlockSpec((TT, C), lambda i: (i, 0)),
            pl.BlockSpec((L * G, D, K), lambda i: (0, 0, 0)),
            pl.BlockSpec((L * G, K, D), lambda i: (0, 0, 0)),
        ],
        out_specs=pl.BlockSpec((TT, C), lambda i: (i, 0)),
        out_shape=jax.ShapeDtypeStruct((T, C), z.dtype),
    )(z2, cbT, cbR)
    q = quant.reshape(B, N, C)
    return z + jax.lax.stop_gradient(q - z)
